# Initial kernel scaffold; baseline (speedup 1.0000x reference)
import jax
import jax.numpy as jnp
from jax.experimental import pallas as pl


def kernel(x, codebook):
    raise NotImplementedError("write your pallas kernel here")


# speedup vs baseline: 1.1854x; 1.1854x over previous
import functools

import jax
import jax.numpy as jnp
from jax import lax
from jax.experimental import pallas as pl
from jax.experimental.pallas import tpu as pltpu
from jax.experimental.pallas import tpu_sc as plsc

_K = 1024
_D = 64
_ROWS = 18432
_R = 1024
_NBLK = _ROWS // _R

_NW = 32
_BPW = _ROWS // _NW
_CH = 96
_NCH = _BPW // _CH


def _argmin_body(x_ref, cb_ref, cn_ref, rsq_ref, idx_ref, loss_ref):
    xb = x_ref[...]
    cb = cb_ref[...]
    s = lax.dot_general(xb, cb, (((1,), (1,)), ((), ())),
                        preferred_element_type=jnp.float32)
    d = (rsq_ref[...] - 2.0 * s) + cn_ref[...]
    m = jnp.min(d, axis=1, keepdims=True)
    ii = lax.broadcasted_iota(jnp.int32, d.shape, 1)
    idx = jnp.min(jnp.where(d == m, ii, _K), axis=1)
    idx_ref[...] = idx.reshape(_R // 128, 128)

    @pl.when(pl.program_id(0) == 0)
    def _():
        loss_ref[...] = jnp.zeros((1, 1), jnp.float32)

    loss_ref[...] += jnp.sum(m, axis=(0, 1), keepdims=True)


_argmin_call = pl.pallas_call(
    _argmin_body,
    grid=(_NBLK,),
    in_specs=[
        pl.BlockSpec((_R, _D), lambda i: (i, 0)),
        pl.BlockSpec((_K, _D), lambda i: (0, 0)),
        pl.BlockSpec((1, _K), lambda i: (0, 0)),
        pl.BlockSpec((_R, 1), lambda i: (i, 0)),
    ],
    out_specs=[
        pl.BlockSpec((_R // 128, 128), lambda i: (i, 0)),
        pl.BlockSpec((1, 1), lambda i: (0, 0)),
    ],
    out_shape=[
        jax.ShapeDtypeStruct((_ROWS // 128, 128), jnp.int32),
        jax.ShapeDtypeStruct((1, 1), jnp.float32),
    ],
)


@functools.partial(
    pl.kernel,
    mesh=plsc.VectorSubcoreMesh(core_axis_name="c", subcore_axis_name="s"),
    compiler_params=pltpu.CompilerParams(use_tc_tiling_on_sc=False),
    out_type=jax.ShapeDtypeStruct((_ROWS, _D), jnp.float32),
    scratch_types=[
        pltpu.VMEM((_NCH, _CH), jnp.int32),
        pltpu.VMEM((_BPW, _D), jnp.float32),
        pltpu.SemaphoreType.DMA,
    ],
)
def _sc_gather(cb_hbm, idx_hbm, out_hbm, idx_v, rows_v, sem):
    wid = lax.axis_index("c") * 16 + lax.axis_index("s")
    pltpu.sync_copy(idx_hbm.at[wid], idx_v)
    copies = [
        pltpu.async_copy(cb_hbm.at[idx_v.at[j]],
                         rows_v.at[pl.ds(j * _CH, _CH)], sem)
        for j in range(_NCH)
    ]
    for c in copies:
        c.wait()
    pltpu.sync_copy(rows_v, out_hbm.at[pl.ds(wid * _BPW, _BPW)])


def kernel(x, codebook):
    B, T, D = x.shape
    flat = x.reshape(-1, D)
    cn = jnp.sum(codebook ** 2, axis=1)[None, :]
    rsq = jnp.sum(flat ** 2, axis=1, keepdims=True)
    idx2d, loss_sum = _argmin_call(flat, codebook, cn, rsq)
    idx_flat = idx2d.reshape(-1)
    zq = _sc_gather(codebook, idx_flat.reshape(_NW, _NCH, _CH))
    zq_st = zq.reshape(B, T, D)
    loss = 1.25 * loss_sum[0, 0] / (B * T * D)
    return zq_st, loss, idx_flat.reshape(B, T)
